# Initial kernel scaffold; baseline (speedup 1.0000x reference)
import jax
import jax.numpy as jnp
from jax.experimental import pallas as pl


def kernel(x, edge_index, etype, batch, W, b):
    raise NotImplementedError("write your pallas kernel here")


# speedup vs baseline: 3.2166x; 3.2166x over previous
import functools

import jax
import jax.numpy as jnp
from jax import lax
from jax.experimental import pallas as pl
from jax.experimental.pallas import tpu as pltpu
from jax.experimental.pallas import tpu_sc as plsc

N = 10000
D = 256
G = 64
NC = 2
NS = 16
NW = NC * NS
LANES = 16
NB = 1000
NBLK = N // NB


def _sc_hits(src, dst, et, *, ept):
    mesh = plsc.VectorSubcoreMesh(core_axis_name="c", subcore_axis_name="s")

    @functools.partial(
        pl.kernel,
        mesh=mesh,
        out_type=jax.ShapeDtypeStruct((NBLK * NW * NB,), jnp.int32),
        compiler_params=pltpu.CompilerParams(needs_layout_passes=False),
        scratch_types=[
            pltpu.VMEM((ept,), jnp.int32),
            pltpu.VMEM((ept,), jnp.int32),
            pltpu.VMEM((ept,), jnp.int32),
            pltpu.VMEM((N,), jnp.int32),
        ],
    )
    def sc_kernel(src_hbm, dst_hbm, et_hbm, out_hbm, src_v, dst_v, et_v, hits_v):
        wid = lax.axis_index("s") * NC + lax.axis_index("c")
        base = wid * ept
        pltpu.sync_copy(src_hbm.at[pl.ds(base, ept)], src_v)
        pltpu.sync_copy(dst_hbm.at[pl.ds(base, ept)], dst_v)
        pltpu.sync_copy(et_hbm.at[pl.ds(base, ept)], et_v)

        zeros16 = jnp.zeros((LANES,), jnp.int32)

        def zero_body(i, carry):
            hits_v[pl.ds(i * LANES, LANES)] = zeros16
            return carry

        lax.fori_loop(0, N // LANES, zero_body, 0)

        ones16 = jnp.ones((LANES,), jnp.int32)

        def edge_body(j, carry):
            sl = pl.ds(j * LANES, LANES)
            m = et_v[sl] == 1
            plsc.store_scatter(hits_v, [src_v[sl]], ones16, mask=m)
            plsc.store_scatter(hits_v, [dst_v[sl]], ones16, mask=m)
            return carry

        lax.fori_loop(0, ept // LANES, edge_body, 0)

        for k in range(NBLK):
            pltpu.sync_copy(
                hits_v.at[pl.ds(k * NB, NB)],
                out_hbm.at[pl.ds(k * NW * NB + wid * NB, NB)],
            )

    return sc_kernel(src, dst, et)


def _tc_pool(x, batch3, w_row, b2, hits):
    def body(x_ref, batch_ref, w_ref, b_ref, hits_ref, out_ref):
        i = pl.program_id(0)
        xb = x_ref[...]
        logits = jnp.sum(xb * w_ref[...], axis=1) + b_ref[0, 0]
        wgt = 1.0 / (1.0 + jnp.exp(-logits))
        hitsum = jnp.sum(hits_ref[0], axis=0)
        wgt = jnp.where(hitsum > 0, 0.0, wgt)
        bb = batch_ref[0, 0, :]
        gids = lax.broadcasted_iota(jnp.int32, (G, NB), 0)
        sel = jnp.where(gids == bb[None, :], wgt[None, :], 0.0)
        contrib = jnp.dot(sel, xb, preferred_element_type=jnp.float32)

        @pl.when(i == 0)
        def _():
            out_ref[...] = contrib

        @pl.when(i > 0)
        def _():
            out_ref[...] += contrib

    return pl.pallas_call(
        body,
        grid=(NBLK,),
        in_specs=[
            pl.BlockSpec((NB, D), lambda i: (i, 0)),
            pl.BlockSpec((1, 1, NB), lambda i: (i, 0, 0)),
            pl.BlockSpec((1, D), lambda i: (0, 0)),
            pl.BlockSpec((1, 1), lambda i: (0, 0)),
            pl.BlockSpec((1, NW, NB), lambda i: (i, 0, 0)),
        ],
        out_specs=pl.BlockSpec((G, D), lambda i: (0, 0)),
        out_shape=jax.ShapeDtypeStruct((G, D), jnp.float32),
        compiler_params=pltpu.CompilerParams(
            dimension_semantics=("arbitrary",),
        ),
    )(x, batch3, w_row, b2, hits)


def kernel(x, edge_index, etype, batch, W, b):
    e = edge_index.shape[1]
    chunk = NW * LANES
    epad = ((e + chunk - 1) // chunk) * chunk
    ept = epad // NW
    src = jnp.zeros((epad,), jnp.int32).at[:e].set(edge_index[0].astype(jnp.int32))
    dst = jnp.zeros((epad,), jnp.int32).at[:e].set(edge_index[1].astype(jnp.int32))
    et = jnp.zeros((epad,), jnp.int32).at[:e].set(etype.astype(jnp.int32))

    hits = _sc_hits(src, dst, et, ept=ept).reshape(NBLK, NW, NB)

    batch3 = batch.astype(jnp.int32).reshape(NBLK, 1, NB)
    w_row = W.astype(jnp.float32).reshape(1, D)
    b2 = b.astype(jnp.float32).reshape(1, 1)
    return _tc_pool(x, batch3, w_row, b2, hits)


# speedup vs baseline: 4.0016x; 1.2440x over previous
import functools

import jax
import jax.numpy as jnp
from jax import lax
from jax.experimental import pallas as pl
from jax.experimental.pallas import tpu as pltpu
from jax.experimental.pallas import tpu_sc as plsc

N = 10000
D = 256
G = 64
NC = 2
NS = 16
NW = NC * NS
LANES = 16
NB = 1000
NBLK = N // NB


def _sc_hits(ei_flat, et, *, e):
    mesh = plsc.VectorSubcoreMesh(core_axis_name="c", subcore_axis_name="s")

    ept = e // NW
    nfull = ept // LANES
    tail = ept - nfull * LANES
    buf = (nfull + (1 if tail else 0)) * LANES
    UNROLL = 8
    nun = nfull // UNROLL
    rem = nfull - nun * UNROLL

    @functools.partial(
        pl.kernel,
        mesh=mesh,
        out_type=jax.ShapeDtypeStruct((NBLK * NW * NB,), jnp.int32),
        compiler_params=pltpu.CompilerParams(needs_layout_passes=False),
        scratch_types=[
            pltpu.VMEM((buf,), jnp.int32),
            pltpu.VMEM((buf,), jnp.int32),
            pltpu.VMEM((buf,), jnp.int32),
            pltpu.VMEM((N,), jnp.int32),
        ],
    )
    def sc_kernel(ei_hbm, et_hbm, out_hbm, src_v, dst_v, et_v, hits_v):
        wid = lax.axis_index("s") * NC + lax.axis_index("c")
        base = wid * ept
        pltpu.sync_copy(ei_hbm.at[pl.ds(base, ept)], src_v.at[pl.ds(0, ept)])
        pltpu.sync_copy(ei_hbm.at[pl.ds(e + base, ept)], dst_v.at[pl.ds(0, ept)])
        pltpu.sync_copy(et_hbm.at[pl.ds(base, ept)], et_v.at[pl.ds(0, ept)])

        zeros16 = jnp.zeros((LANES,), jnp.int32)
        ZUN = 5

        def zero_body(i, carry):
            for u in range(ZUN):
                hits_v[pl.ds((i * ZUN + u) * LANES, LANES)] = zeros16
            return carry

        lax.fori_loop(0, N // (LANES * ZUN), zero_body, 0)

        ones16 = jnp.ones((LANES,), jnp.int32)

        def scatter_one(j):
            sl = pl.ds(j * LANES, LANES)
            m = et_v[sl] == 1
            plsc.store_scatter(hits_v, [src_v[sl]], ones16, mask=m)
            plsc.store_scatter(hits_v, [dst_v[sl]], ones16, mask=m)

        def edge_body(i, carry):
            for u in range(UNROLL):
                scatter_one(i * UNROLL + u)
            return carry

        lax.fori_loop(0, nun, edge_body, 0)
        for u in range(rem):
            scatter_one(nun * UNROLL + u)
        if tail:
            lanes = lax.iota(jnp.int32, LANES)
            sl = pl.ds(nfull * LANES, LANES)
            m = (et_v[sl] == 1) & (lanes < tail)
            plsc.store_scatter(hits_v, [src_v[sl]], ones16, mask=m)
            plsc.store_scatter(hits_v, [dst_v[sl]], ones16, mask=m)

        for k in range(NBLK):
            pltpu.sync_copy(
                hits_v.at[pl.ds(k * NB, NB)],
                out_hbm.at[pl.ds(k * NW * NB + wid * NB, NB)],
            )

    return sc_kernel(ei_flat, et)


def _tc_pool(x, batch3, w_row, b2, hits):
    def body(x_ref, batch_ref, w_ref, b_ref, hits_ref, out_ref):
        i = pl.program_id(0)
        xb = x_ref[...]
        logits = jnp.sum(xb * w_ref[...], axis=1) + b_ref[0, 0]
        wgt = 1.0 / (1.0 + jnp.exp(-logits))
        hitsum = jnp.sum(hits_ref[0], axis=0)
        wgt = jnp.where(hitsum > 0, 0.0, wgt)
        bb = batch_ref[0, 0, :]
        gids = lax.broadcasted_iota(jnp.int32, (G, NB), 0)
        sel = jnp.where(gids == bb[None, :], wgt[None, :], 0.0)
        contrib = jnp.dot(sel, xb, preferred_element_type=jnp.float32)

        @pl.when(i == 0)
        def _():
            out_ref[...] = contrib

        @pl.when(i > 0)
        def _():
            out_ref[...] += contrib

    return pl.pallas_call(
        body,
        grid=(NBLK,),
        in_specs=[
            pl.BlockSpec((NB, D), lambda i: (i, 0)),
            pl.BlockSpec((1, 1, NB), lambda i: (i, 0, 0)),
            pl.BlockSpec((1, D), lambda i: (0, 0)),
            pl.BlockSpec((1, 1), lambda i: (0, 0)),
            pl.BlockSpec((1, NW, NB), lambda i: (i, 0, 0)),
        ],
        out_specs=pl.BlockSpec((G, D), lambda i: (0, 0)),
        out_shape=jax.ShapeDtypeStruct((G, D), jnp.float32),
        compiler_params=pltpu.CompilerParams(
            dimension_semantics=("arbitrary",),
        ),
    )(x, batch3, w_row, b2, hits)


def kernel(x, edge_index, etype, batch, W, b):
    e = edge_index.shape[1]
    ei_flat = edge_index.astype(jnp.int32).reshape(2 * e)
    et = etype.astype(jnp.int32)

    hits = _sc_hits(ei_flat, et, e=e).reshape(NBLK, NW, NB)

    batch3 = batch.astype(jnp.int32).reshape(NBLK, 1, NB)
    w_row = W.astype(jnp.float32).reshape(1, D)
    b2 = b.astype(jnp.float32).reshape(1, 1)
    return _tc_pool(x, batch3, w_row, b2, hits)


# speedup vs baseline: 4.0244x; 1.0057x over previous
import functools

import jax
import jax.numpy as jnp
from jax import lax
from jax.experimental import pallas as pl
from jax.experimental.pallas import tpu as pltpu
from jax.experimental.pallas import tpu_sc as plsc

N = 10000
D = 256
G = 64
NC = 2
NS = 16
NW = NC * NS
LANES = 16
NB = 1000
NBLK = N // NB
NR = NW + 1


def _sc_hits(ei, et, batch, *, e):
    mesh = plsc.VectorSubcoreMesh(core_axis_name="c", subcore_axis_name="s")

    CH = 128
    nch = e // CH
    qc = nch // NW
    rc = nch % NW
    buf_hi = (qc + 1) * CH
    buf_lo = qc * CH
    UNROLL = 8

    @functools.partial(
        pl.kernel,
        mesh=mesh,
        out_type=jax.ShapeDtypeStruct((NBLK * NR * NB,), jnp.int32),
        compiler_params=pltpu.CompilerParams(needs_layout_passes=False),
        scratch_types=[
            pltpu.VMEM((2, buf_hi), jnp.int32),
            pltpu.VMEM((2, buf_lo), jnp.int32),
            pltpu.VMEM((buf_hi,), jnp.int32),
            pltpu.VMEM((N,), jnp.int32),
            pltpu.VMEM((NB,), jnp.int32),
        ],
    )
    def sc_kernel(ei_hbm, et_hbm, b_hbm, out_hbm, ei_hi, ei_lo, et_v, hits_v, b_v):
        wid = lax.axis_index("s") * NC + lax.axis_index("c")
        base = (wid * qc + jnp.minimum(wid, rc)) * CH

        zeros16 = jnp.zeros((LANES,), jnp.int32)
        ZUN = 5

        def zero_body(i, carry):
            for u in range(ZUN):
                hits_v[pl.ds((i * ZUN + u) * LANES, LANES)] = zeros16
            return carry

        lax.fori_loop(0, N // (LANES * ZUN), zero_body, 0)

        ones16 = jnp.ones((LANES,), jnp.int32)

        def work(ei_v, nchunks):
            ept = nchunks * CH
            nvec = ept // LANES
            pltpu.sync_copy(ei_hbm.at[:, pl.ds(base, ept)], ei_v)
            pltpu.sync_copy(et_hbm.at[pl.ds(base, ept)], et_v.at[pl.ds(0, ept)])

            def scatter_one(j):
                sl = pl.ds(j * LANES, LANES)
                m = et_v[sl] == 1
                plsc.store_scatter(hits_v, [ei_v[0, sl]], ones16, mask=m)
                plsc.store_scatter(hits_v, [ei_v[1, sl]], ones16, mask=m)

            def edge_body(i, carry):
                for u in range(UNROLL):
                    scatter_one(i * UNROLL + u)
                return carry

            nun = nvec // UNROLL
            lax.fori_loop(0, nun, edge_body, 0)
            for u in range(nvec - nun * UNROLL):
                scatter_one(nun * UNROLL + u)

        if rc:
            @pl.when(wid < rc)
            def _():
                work(ei_hi, qc + 1)

            @pl.when(wid >= rc)
            def _():
                work(ei_lo, qc)
        else:
            work(ei_lo, qc)

        for k in range(NBLK):
            pltpu.sync_copy(
                hits_v.at[pl.ds(k * NB, NB)],
                out_hbm.at[pl.ds(k * NR * NB + wid * NB, NB)],
            )

        @pl.when(wid < NBLK)
        def _():
            pltpu.sync_copy(b_hbm.at[pl.ds(wid * NB, NB)], b_v)
            pltpu.sync_copy(b_v, out_hbm.at[pl.ds(wid * NR * NB + NW * NB, NB)])

    return sc_kernel(ei, et, batch)


def _tc_pool(x, w_row, b2, hb):
    def body(x_ref, w_ref, b_ref, hb_ref, out_ref):
        i = pl.program_id(0)
        xb = x_ref[...]
        logits = jnp.sum(xb * w_ref[...], axis=1) + b_ref[0, 0]
        wgt = 1.0 / (1.0 + jnp.exp(-logits))
        hitsum = jnp.sum(hb_ref[0, :NW, :], axis=0)
        wgt = jnp.where(hitsum > 0, 0.0, wgt)
        bb = hb_ref[0, NW, :]
        gids = lax.broadcasted_iota(jnp.int32, (G, NB), 0)
        sel = jnp.where(gids == bb[None, :], wgt[None, :], 0.0)
        contrib = jnp.dot(sel, xb, preferred_element_type=jnp.float32)

        @pl.when(i == 0)
        def _():
            out_ref[...] = contrib

        @pl.when(i > 0)
        def _():
            out_ref[...] += contrib

    return pl.pallas_call(
        body,
        grid=(NBLK,),
        in_specs=[
            pl.BlockSpec((NB, D), lambda i: (i, 0)),
            pl.BlockSpec((1, D), lambda i: (0, 0)),
            pl.BlockSpec((1, 1), lambda i: (0, 0)),
            pl.BlockSpec((1, NR, NB), lambda i: (i, 0, 0)),
        ],
        out_specs=pl.BlockSpec((G, D), lambda i: (0, 0)),
        out_shape=jax.ShapeDtypeStruct((G, D), jnp.float32),
        compiler_params=pltpu.CompilerParams(
            dimension_semantics=("arbitrary",),
        ),
    )(x, w_row, b2, hb)


def kernel(x, edge_index, etype, batch, W, b):
    e = edge_index.shape[1]
    hb = _sc_hits(
        edge_index.astype(jnp.int32),
        etype.astype(jnp.int32),
        batch.astype(jnp.int32),
        e=e,
    ).reshape(NBLK, NR, NB)

    w_row = W.astype(jnp.float32).reshape(1, D)
    b2 = b.astype(jnp.float32).reshape(1, 1)
    return _tc_pool(x, w_row, b2, hb)
